# baseline (speedup 1.0000x reference)
import functools

import jax
import jax.numpy as jnp
from jax import lax
from jax.experimental import pallas as pl
from jax.experimental.pallas import tpu as pltpu
from jax.experimental.pallas import tpu_sc as plsc

BATCH = 16384
VOCAB_N = 1000000
DIM = 64
LANES = 16
NUM_CORES = 2
NUM_SUBCORES = 16
NW = NUM_CORES * NUM_SUBCORES
SSHIFT = 15
STRIPE = 1 << SSHIFT
BLKW = 512
SWEEP_END = (VOCAB_N // BLKW) * BLKW
TAILW = 128
TAILB = VOCAB_N - TAILW
LASTW = TAILB >> SSHIFT
CAP = BATCH + 128

_mesh = plsc.VectorSubcoreMesh(core_axis_name="c", subcore_axis_name="s")


def _match_extract(buf, base, width, mc, my_idx, my_pos, mati, matp,
                   colbuf, stag, sem_o):
    lanes = lax.iota(jnp.int32, LANES)
    nch = (mc + LANES - 1) // LANES

    def chunk(k, carry):
        off = k * LANES
        vi = my_idx[pl.ds(off, LANES)]
        vp = my_pos[pl.ds(off, LANES)]
        valid = lanes < (mc - off)
        inb = valid & (vi >= base) & (vi < base + width)
        nm = plsc.all_reduce_population_count(inb)[0]

        @pl.when(nm > 0)
        def _():
            prefm = plsc.cumsum(jnp.where(inb, jnp.full((LANES,), 1, jnp.int32), jnp.full((LANES,), 0, jnp.int32)))
            mtgt = jnp.where(inb, prefm - 1, 0)
            plsc.store_scatter(mati, [mtgt], vi, mask=inb)
            plsc.store_scatter(matp, [mtgt], vp, mask=inb)
            mvi = mati[...]
            mvp = matp[...]
            cols = jnp.where(lanes < nm, mvi - base, 0)
            for d in range(DIM):
                vals = plsc.load_gather(
                    buf, [jnp.full((LANES,), d, jnp.int32), cols])
                plsc.store_scatter(colbuf, [lanes * (2 * DIM) + d], vals)
            for l in range(LANES):
                @pl.when(l < nm)
                def _():
                    pltpu.async_copy(colbuf.at[pl.ds(l * 2 * DIM, 2 * DIM)],
                                     stag.at[pl.ds(mvp[l] * (2 * DIM),
                                                   2 * DIM)], sem_o)

            def drain(_l, c2):
                pltpu.make_async_copy(stag.at[pl.ds(0, 2 * DIM)],
                                      colbuf.at[pl.ds(0, 2 * DIM)], sem_o).wait()
                return c2

            lax.fori_loop(0, nm, drain, 0)

        return carry

    lax.fori_loop(0, nch, chunk, 0)


def _serve(w, idx_hbm, tab_hbm, tail_hbm, stag, idxall, my_idx, my_pos,
           blockbuf, tailbuf, colbuf, mati, matp, sem_o):
    pltpu.sync_copy(idx_hbm, idxall)

    lanes = lax.iota(jnp.int32, LANES)

    def fbody(k, cnt):
        v = idxall[pl.ds(k * LANES, LANES)]
        m = (v >> SSHIFT) == w
        pos = k * LANES + lanes
        pref = plsc.cumsum(jnp.where(m, jnp.full((LANES,), 1, jnp.int32), jnp.full((LANES,), 0, jnp.int32)))
        tgt = cnt + pref - 1
        stgt = jnp.where(m, tgt, 0)
        plsc.store_scatter(my_idx, [stgt], v, mask=m)
        plsc.store_scatter(my_pos, [stgt], pos, mask=m)
        return cnt + plsc.all_reduce_population_count(m)[0]

    mc = lax.fori_loop(0, BATCH // LANES, fbody, 0)

    lo = w * STRIPE
    nb = jnp.where(w == LASTW, (SWEEP_END - LASTW * STRIPE) // BLKW,
                   jnp.where(w > LASTW, 0, STRIPE // BLKW))

    def bbody(b, carry):
        tb = lo + b * BLKW
        tbh = pl.multiple_of(tb, BLKW)
        pltpu.sync_copy(tab_hbm.at[:, pl.ds(tbh, BLKW)], blockbuf)
        _match_extract(blockbuf, tb, BLKW, mc, my_idx, my_pos, mati, matp,
                       colbuf, stag, sem_o)
        return carry

    lax.fori_loop(0, nb, bbody, 0)

    @pl.when(w == LASTW)
    def _():
        pltpu.sync_copy(tail_hbm, tailbuf)
        _match_extract(tailbuf, TAILB, TAILW, mc, my_idx, my_pos, mati, matp,
                       colbuf, stag, sem_o)


@functools.partial(
    pl.kernel,
    mesh=_mesh,
    compiler_params=pltpu.CompilerParams(needs_layout_passes=False),
    out_type=(jax.ShapeDtypeStruct((BATCH * 2 * DIM,), jnp.float32),
              jax.ShapeDtypeStruct((BATCH * 2 * DIM,), jnp.float32)),
    scratch_types=[
        pltpu.VMEM((BATCH,), jnp.int32),
        pltpu.VMEM((CAP,), jnp.int32),
        pltpu.VMEM((CAP,), jnp.int32),
        pltpu.VMEM((DIM, BLKW), jnp.float32),
        pltpu.VMEM((DIM, TAILW), jnp.float32),
        pltpu.VMEM((LANES * 2 * DIM,), jnp.float32),
        pltpu.VMEM((LANES,), jnp.int32),
        pltpu.VMEM((LANES,), jnp.int32),
        pltpu.SemaphoreType.DMA,
    ],
)
def _gmf_gather(user_hbm, item_hbm, utab_hbm, itab_hbm, tail_u, tail_i,
                stag_u, stag_i,
                idxall, my_idx, my_pos, blockbuf, tailbuf, colbuf,
                mati, matp, sem_o):
    w = lax.axis_index("c") * NUM_SUBCORES + lax.axis_index("s")
    _serve(w, user_hbm, utab_hbm, tail_u, stag_u, idxall, my_idx, my_pos,
           blockbuf, tailbuf, colbuf, mati, matp, sem_o)
    _serve(w, item_hbm, itab_hbm, tail_i, stag_i, idxall, my_idx, my_pos,
           blockbuf, tailbuf, colbuf, mati, matp, sem_o)


_HB = 128
_RW = 2 * DIM


@functools.partial(
    pl.kernel,
    mesh=_mesh,
    compiler_params=pltpu.CompilerParams(needs_layout_passes=False),
    out_type=jax.ShapeDtypeStruct((BATCH * DIM,), jnp.float32),
    scratch_types=[
        pltpu.VMEM((_HB * _RW,), jnp.float32),
        pltpu.VMEM((_HB * _RW,), jnp.float32),
        pltpu.VMEM((_HB * DIM,), jnp.float32),
    ],
)
def _gmf_mul(su_hbm, si_hbm, out_hbm, ubuf, ibuf, obuf):
    wid = lax.axis_index("s") * NUM_CORES + lax.axis_index("c")
    base = wid * (BATCH // NW)

    for h in range(BATCH // NW // _HB):
        hb = base + h * _HB
        pltpu.sync_copy(su_hbm.at[pl.ds(hb * _RW, _HB * _RW)], ubuf)
        pltpu.sync_copy(si_hbm.at[pl.ds(hb * _RW, _HB * _RW)], ibuf)

        def rbody(r, carry):
            for g in range(DIM // LANES):
                so = pl.ds(r * _RW + g * LANES, LANES)
                do = pl.ds(r * DIM + g * LANES, LANES)
                obuf[do] = ubuf[so] * ibuf[so]
            return carry

        lax.fori_loop(0, _HB, rbody, 0)
        pltpu.sync_copy(obuf, out_hbm.at[pl.ds(hb * DIM, _HB * DIM)])


def kernel(user, item, user_table, item_table):
    tail_u = user_table[VOCAB_N - TAILW:].T
    tail_i = item_table[VOCAB_N - TAILW:].T
    su, si = _gmf_gather(user, item, user_table.T, item_table.T,
                         tail_u, tail_i)
    out1 = _gmf_mul(su, si)
    return out1.reshape(BATCH, DIM)


# speedup vs baseline: 2.2174x; 2.2174x over previous
import functools

import jax
import jax.numpy as jnp
from jax import lax
from jax.experimental import pallas as pl
from jax.experimental.pallas import tpu as pltpu
from jax.experimental.pallas import tpu_sc as plsc

BATCH = 16384
VOCAB_N = 1000000
DIM = 64
LANES = 16
NUM_CORES = 2
NUM_SUBCORES = 16
NW = NUM_CORES * NUM_SUBCORES
SSHIFT = 15
STRIPE = 1 << SSHIFT
BLKW = 512
SWEEP_END = (VOCAB_N // BLKW) * BLKW
TAILW = 128
TAILB = VOCAB_N - TAILW
LASTW = TAILB >> SSHIFT
CAP = BATCH + 128

_mesh = plsc.VectorSubcoreMesh(core_axis_name="c", subcore_axis_name="s")


NSLOT = 6
_ONES = None


def _sweep_window(buf, base, width, mc, my_idx, my_pos, pend_col, pend_pos,
                  colbuf, stag, sem_o):
    lanes = lax.iota(jnp.int32, LANES)
    ones = jnp.full((LANES,), 1, jnp.int32)
    zeros = jnp.full((LANES,), 0, jnp.int32)
    nch4 = (mc + 4 * LANES - 1) // (4 * LANES)

    def scan4(k4, bc):
        for j in range(4):
            off = (k4 * 4 + j) * LANES
            vi = my_idx[pl.ds(off, LANES)]
            vp = my_pos[pl.ds(off, LANES)]
            valid = lanes < (mc - off)
            inb = valid & (vi >= base) & (vi < base + width)
            pref = plsc.cumsum(jnp.where(inb, ones, zeros))
            stgt = jnp.where(inb, bc + pref - 1, 0)
            plsc.store_scatter(pend_col, [stgt], vi - base, mask=inb)
            plsc.store_scatter(pend_pos, [stgt], vp, mask=inb)
            bc = bc + plsc.all_reduce_population_count(inb)[0]
        return bc

    bc = lax.fori_loop(0, nch4, scan4, 0)
    ng = (bc + LANES - 1) // LANES

    def egroup(g, st):
        drained, issued = st
        ndrain = jnp.where((g > 0) & (lax.rem(g, NSLOT) == 0),
                           issued - drained, 0)

        def dwait(_l, c2):
            pltpu.make_async_copy(stag.at[pl.ds(0, 2 * DIM)],
                                  colbuf.at[pl.ds(0, 2 * DIM)], sem_o).wait()
            return c2

        lax.fori_loop(0, ndrain, dwait, 0)
        drained = drained + ndrain

        goff = g * LANES
        rem = bc - goff
        gvi = pend_col[pl.ds(goff, LANES)]
        gvp = pend_pos[pl.ds(goff, LANES)]
        cols = jnp.where(lanes < rem, gvi, 0)
        slot = lax.rem(g, NSLOT) * (LANES * 2 * DIM)
        for d in range(DIM):
            vals = plsc.load_gather(
                buf, [jnp.full((LANES,), d, jnp.int32), cols])
            plsc.store_scatter(colbuf, [slot + lanes * (2 * DIM) + d], vals)
        for l in range(LANES):
            @pl.when(l < rem)
            def _():
                pltpu.async_copy(
                    colbuf.at[pl.ds(slot + l * 2 * DIM, 2 * DIM)],
                    stag.at[pl.ds(gvp[l] * (2 * DIM), 2 * DIM)], sem_o)
        issued = issued + jnp.minimum(rem, LANES)
        return drained, issued

    drained, issued = lax.fori_loop(0, ng, egroup, (0, 0))

    def dwait2(_l, c2):
        pltpu.make_async_copy(stag.at[pl.ds(0, 2 * DIM)],
                              colbuf.at[pl.ds(0, 2 * DIM)], sem_o).wait()
        return c2

    lax.fori_loop(0, issued - drained, dwait2, 0)


def _serve(w, idx_hbm, tab_hbm, tail_hbm, stag, idxall, my_idx, my_pos,
           pend_col, pend_pos, blockbuf, colbuf, sem_o):
    pltpu.sync_copy(idx_hbm, idxall)

    lanes = lax.iota(jnp.int32, LANES)
    ones = jnp.full((LANES,), 1, jnp.int32)
    zeros = jnp.full((LANES,), 0, jnp.int32)

    def fbody(k4, cnt):
        for j in range(4):
            k = k4 * 4 + j
            v = idxall[pl.ds(k * LANES, LANES)]
            m = (v >> SSHIFT) == w
            pos = k * LANES + lanes
            pref = plsc.cumsum(jnp.where(m, ones, zeros))
            stgt = jnp.where(m, cnt + pref - 1, 0)
            plsc.store_scatter(my_idx, [stgt], v, mask=m)
            plsc.store_scatter(my_pos, [stgt], pos, mask=m)
            cnt = cnt + plsc.all_reduce_population_count(m)[0]
        return cnt

    mc = lax.fori_loop(0, BATCH // LANES // 4, fbody, 0)

    lo = w * STRIPE
    nb = jnp.where(w == LASTW, (SWEEP_END - LASTW * STRIPE) // BLKW,
                   jnp.where(w > LASTW, 0, STRIPE // BLKW))

    def bbody(b, carry):
        tb = lo + b * BLKW
        tbh = pl.multiple_of(tb, BLKW)
        pltpu.sync_copy(tab_hbm.at[:, pl.ds(tbh, BLKW)], blockbuf)
        _sweep_window(blockbuf, tb, BLKW, mc, my_idx, my_pos,
                      pend_col, pend_pos, colbuf, stag, sem_o)
        return carry

    lax.fori_loop(0, nb, bbody, 0)

    @pl.when(w == LASTW)
    def _():
        pltpu.sync_copy(tail_hbm, blockbuf.at[:, pl.ds(0, TAILW)])
        _sweep_window(blockbuf, TAILB, TAILW, mc, my_idx, my_pos,
                      pend_col, pend_pos, colbuf, stag, sem_o)


@functools.partial(
    pl.kernel,
    mesh=_mesh,
    compiler_params=pltpu.CompilerParams(needs_layout_passes=False),
    out_type=(jax.ShapeDtypeStruct((BATCH * 2 * DIM,), jnp.float32),
              jax.ShapeDtypeStruct((BATCH * 2 * DIM,), jnp.float32)),
    scratch_types=[
        pltpu.VMEM((BATCH,), jnp.int32),
        pltpu.VMEM((CAP,), jnp.int32),
        pltpu.VMEM((CAP,), jnp.int32),
        pltpu.VMEM((BATCH,), jnp.int32),
        pltpu.VMEM((BATCH,), jnp.int32),
        pltpu.VMEM((DIM, BLKW), jnp.float32),
        pltpu.VMEM((NSLOT * LANES * 2 * DIM,), jnp.float32),
        pltpu.SemaphoreType.DMA,
    ],
)
def _gmf_gather(user_hbm, item_hbm, utab_hbm, itab_hbm, tail_u, tail_i,
                stag_u, stag_i,
                idxall, my_idx, my_pos, pend_col, pend_pos, blockbuf,
                colbuf, sem_o):
    w = lax.axis_index("c") * NUM_SUBCORES + lax.axis_index("s")
    _serve(w, user_hbm, utab_hbm, tail_u, stag_u, idxall, my_idx, my_pos,
           pend_col, pend_pos, blockbuf, colbuf, sem_o)
    _serve(w, item_hbm, itab_hbm, tail_i, stag_i, idxall, my_idx, my_pos,
           pend_col, pend_pos, blockbuf, colbuf, sem_o)


_HB = 128
_RW = 2 * DIM


@functools.partial(
    pl.kernel,
    mesh=_mesh,
    compiler_params=pltpu.CompilerParams(needs_layout_passes=False),
    out_type=jax.ShapeDtypeStruct((BATCH * DIM,), jnp.float32),
    scratch_types=[
        pltpu.VMEM((_HB * _RW,), jnp.float32),
        pltpu.VMEM((_HB * _RW,), jnp.float32),
        pltpu.VMEM((_HB * DIM,), jnp.float32),
    ],
)
def _gmf_mul(su_hbm, si_hbm, out_hbm, ubuf, ibuf, obuf):
    wid = lax.axis_index("s") * NUM_CORES + lax.axis_index("c")
    base = wid * (BATCH // NW)

    for h in range(BATCH // NW // _HB):
        hb = base + h * _HB
        pltpu.sync_copy(su_hbm.at[pl.ds(hb * _RW, _HB * _RW)], ubuf)
        pltpu.sync_copy(si_hbm.at[pl.ds(hb * _RW, _HB * _RW)], ibuf)

        def rbody(r, carry):
            for g in range(DIM // LANES):
                so = pl.ds(r * _RW + g * LANES, LANES)
                do = pl.ds(r * DIM + g * LANES, LANES)
                obuf[do] = ubuf[so] * ibuf[so]
            return carry

        lax.fori_loop(0, _HB, rbody, 0)
        pltpu.sync_copy(obuf, out_hbm.at[pl.ds(hb * DIM, _HB * DIM)])


def kernel(user, item, user_table, item_table):
    tail_u = user_table[VOCAB_N - TAILW:].T
    tail_i = item_table[VOCAB_N - TAILW:].T
    su, si = _gmf_gather(user, item, user_table.T, item_table.T,
                         tail_u, tail_i)
    out1 = _gmf_mul(su, si)
    return out1.reshape(BATCH, DIM)


# speedup vs baseline: 2.4110x; 1.0873x over previous
import functools

import jax
import jax.numpy as jnp
from jax import lax
from jax.experimental import pallas as pl
from jax.experimental.pallas import tpu as pltpu
from jax.experimental.pallas import tpu_sc as plsc

BATCH = 16384
VOCAB_N = 1000000
DIM = 64
LANES = 16
NUM_CORES = 2
NUM_SUBCORES = 16
NW = NUM_CORES * NUM_SUBCORES
SSHIFT = 15
STRIPE = 1 << SSHIFT
BLKW = 256
SWEEP_END = (VOCAB_N // BLKW) * BLKW
TAILW = 128
TAILB = VOCAB_N - TAILW
LASTW = TAILB >> SSHIFT
CAP = BATCH + 128

_mesh = plsc.VectorSubcoreMesh(core_axis_name="c", subcore_axis_name="s")


NSLOT = 6
_ONES = None


def _sweep_window(buf, base, width, mc, my_idx, my_pos, pend_col, pend_pos,
                  colbuf, stag, sem_o):
    lanes = lax.iota(jnp.int32, LANES)
    ones = jnp.full((LANES,), 1, jnp.int32)
    zeros = jnp.full((LANES,), 0, jnp.int32)
    nch4 = (mc + 4 * LANES - 1) // (4 * LANES)

    def scan4(k4, bc):
        for j in range(4):
            off = (k4 * 4 + j) * LANES
            vi = my_idx[pl.ds(off, LANES)]
            vp = my_pos[pl.ds(off, LANES)]
            valid = lanes < (mc - off)
            inb = valid & (vi >= base) & (vi < base + width)
            pref = plsc.cumsum(jnp.where(inb, ones, zeros))
            stgt = jnp.where(inb, bc + pref - 1, 0)
            plsc.store_scatter(pend_col, [stgt], vi - base, mask=inb)
            plsc.store_scatter(pend_pos, [stgt], vp, mask=inb)
            bc = bc + plsc.all_reduce_population_count(inb)[0]
        return bc

    bc = lax.fori_loop(0, nch4, scan4, 0)
    ng = (bc + LANES - 1) // LANES

    def egroup(g, st):
        drained, issued = st
        ndrain = jnp.where((g > 0) & (lax.rem(g, NSLOT) == 0),
                           issued - drained, 0)

        def dwait(_l, c2):
            pltpu.make_async_copy(stag.at[pl.ds(0, 2 * DIM)],
                                  colbuf.at[pl.ds(0, 2 * DIM)], sem_o).wait()
            return c2

        lax.fori_loop(0, ndrain, dwait, 0)
        drained = drained + ndrain

        goff = g * LANES
        rem = bc - goff
        gvi = pend_col[pl.ds(goff, LANES)]
        gvp = pend_pos[pl.ds(goff, LANES)]
        cols = jnp.where(lanes < rem, gvi, 0)
        slot = lax.rem(g, NSLOT) * (LANES * 2 * DIM)
        for d in range(DIM):
            vals = plsc.load_gather(
                buf, [jnp.full((LANES,), d, jnp.int32), cols])
            plsc.store_scatter(colbuf, [slot + lanes * (2 * DIM) + d], vals)
        for l in range(LANES):
            @pl.when(l < rem)
            def _():
                pltpu.async_copy(
                    colbuf.at[pl.ds(slot + l * 2 * DIM, 2 * DIM)],
                    stag.at[pl.ds(gvp[l] * (2 * DIM), 2 * DIM)], sem_o)
        issued = issued + jnp.minimum(rem, LANES)
        return drained, issued

    drained, issued = lax.fori_loop(0, ng, egroup, (0, 0))

    def dwait2(_l, c2):
        pltpu.make_async_copy(stag.at[pl.ds(0, 2 * DIM)],
                              colbuf.at[pl.ds(0, 2 * DIM)], sem_o).wait()
        return c2

    lax.fori_loop(0, issued - drained, dwait2, 0)


def _serve(w, idx_hbm, tab_hbm, tail_hbm, stag, idxall, my_idx, my_pos,
           pend_col, pend_pos, blockbuf0, blockbuf1, colbuf, sem_o, sem_b):
    pltpu.sync_copy(idx_hbm, idxall)

    lanes = lax.iota(jnp.int32, LANES)
    ones = jnp.full((LANES,), 1, jnp.int32)
    zeros = jnp.full((LANES,), 0, jnp.int32)

    def fbody(k4, cnt):
        for j in range(4):
            k = k4 * 4 + j
            v = idxall[pl.ds(k * LANES, LANES)]
            m = (v >> SSHIFT) == w
            pos = k * LANES + lanes
            pref = plsc.cumsum(jnp.where(m, ones, zeros))
            stgt = jnp.where(m, cnt + pref - 1, 0)
            plsc.store_scatter(my_idx, [stgt], v, mask=m)
            plsc.store_scatter(my_pos, [stgt], pos, mask=m)
            cnt = cnt + plsc.all_reduce_population_count(m)[0]
        return cnt

    mc = lax.fori_loop(0, BATCH // LANES // 4, fbody, 0)

    lo = w * STRIPE
    nb = jnp.where(w == LASTW, (SWEEP_END - LASTW * STRIPE) // BLKW,
                   jnp.where(w > LASTW, 0, STRIPE // BLKW))

    def _fetch(i, buf):
        tb = jnp.minimum(lo + i * BLKW, SWEEP_END - BLKW)
        pltpu.async_copy(tab_hbm.at[:, pl.ds(pl.multiple_of(tb, BLKW), BLKW)],
                         buf, sem_b)

    def _bwait(buf):
        pltpu.make_async_copy(tab_hbm.at[:, pl.ds(0, BLKW)], buf, sem_b).wait()

    _fetch(0, blockbuf0)

    def bbody(b2, carry):
        i0 = 2 * b2
        i1 = i0 + 1
        _bwait(blockbuf0)
        _fetch(i1, blockbuf1)
        _sweep_window(blockbuf0, lo + i0 * BLKW, BLKW, mc, my_idx, my_pos,
                      pend_col, pend_pos, colbuf, stag, sem_o)
        _bwait(blockbuf1)
        _fetch(i0 + 2, blockbuf0)

        @pl.when(i1 < nb)
        def _():
            _sweep_window(blockbuf1, lo + i1 * BLKW, BLKW, mc, my_idx, my_pos,
                          pend_col, pend_pos, colbuf, stag, sem_o)

        return carry

    lax.fori_loop(0, (nb + 1) // 2, bbody, 0)
    _bwait(blockbuf0)

    @pl.when(w == LASTW)
    def _():
        pltpu.sync_copy(tail_hbm, blockbuf0.at[:, pl.ds(0, TAILW)])
        _sweep_window(blockbuf0, TAILB, TAILW, mc, my_idx, my_pos,
                      pend_col, pend_pos, colbuf, stag, sem_o)


@functools.partial(
    pl.kernel,
    mesh=_mesh,
    compiler_params=pltpu.CompilerParams(needs_layout_passes=False),
    out_type=(jax.ShapeDtypeStruct((BATCH * 2 * DIM,), jnp.float32),
              jax.ShapeDtypeStruct((BATCH * 2 * DIM,), jnp.float32)),
    scratch_types=[
        pltpu.VMEM((BATCH,), jnp.int32),
        pltpu.VMEM((CAP,), jnp.int32),
        pltpu.VMEM((CAP,), jnp.int32),
        pltpu.VMEM((BATCH,), jnp.int32),
        pltpu.VMEM((BATCH,), jnp.int32),
        pltpu.VMEM((DIM, BLKW), jnp.float32),
        pltpu.VMEM((DIM, BLKW), jnp.float32),
        pltpu.VMEM((NSLOT * LANES * 2 * DIM,), jnp.float32),
        pltpu.SemaphoreType.DMA,
        pltpu.SemaphoreType.DMA,
    ],
)
def _gmf_gather(user_hbm, item_hbm, utab_hbm, itab_hbm, tail_u, tail_i,
                stag_u, stag_i,
                idxall, my_idx, my_pos, pend_col, pend_pos, blockbuf0,
                blockbuf1, colbuf, sem_o, sem_b):
    w = lax.axis_index("c") * NUM_SUBCORES + lax.axis_index("s")
    _serve(w, user_hbm, utab_hbm, tail_u, stag_u, idxall, my_idx, my_pos,
           pend_col, pend_pos, blockbuf0, blockbuf1, colbuf, sem_o, sem_b)
    _serve(w, item_hbm, itab_hbm, tail_i, stag_i, idxall, my_idx, my_pos,
           pend_col, pend_pos, blockbuf0, blockbuf1, colbuf, sem_o, sem_b)


_HB = 128
_RW = 2 * DIM


@functools.partial(
    pl.kernel,
    mesh=_mesh,
    compiler_params=pltpu.CompilerParams(needs_layout_passes=False),
    out_type=jax.ShapeDtypeStruct((BATCH * DIM,), jnp.float32),
    scratch_types=[
        pltpu.VMEM((_HB * _RW,), jnp.float32),
        pltpu.VMEM((_HB * _RW,), jnp.float32),
        pltpu.VMEM((_HB * DIM,), jnp.float32),
    ],
)
def _gmf_mul(su_hbm, si_hbm, out_hbm, ubuf, ibuf, obuf):
    wid = lax.axis_index("s") * NUM_CORES + lax.axis_index("c")
    base = wid * (BATCH // NW)

    for h in range(BATCH // NW // _HB):
        hb = base + h * _HB
        pltpu.sync_copy(su_hbm.at[pl.ds(hb * _RW, _HB * _RW)], ubuf)
        pltpu.sync_copy(si_hbm.at[pl.ds(hb * _RW, _HB * _RW)], ibuf)

        def rbody(r, carry):
            for g in range(DIM // LANES):
                so = pl.ds(r * _RW + g * LANES, LANES)
                do = pl.ds(r * DIM + g * LANES, LANES)
                obuf[do] = ubuf[so] * ibuf[so]
            return carry

        lax.fori_loop(0, _HB, rbody, 0)
        pltpu.sync_copy(obuf, out_hbm.at[pl.ds(hb * DIM, _HB * DIM)])


def kernel(user, item, user_table, item_table):
    tail_u = user_table[VOCAB_N - TAILW:].T
    tail_i = item_table[VOCAB_N - TAILW:].T
    su, si = _gmf_gather(user, item, user_table.T, item_table.T,
                         tail_u, tail_i)
    out1 = _gmf_mul(su, si)
    return out1.reshape(BATCH, DIM)


# speedup vs baseline: 2.9614x; 1.2283x over previous
import functools

import jax
import jax.numpy as jnp
from jax import lax
from jax.experimental import pallas as pl
from jax.experimental.pallas import tpu as pltpu
from jax.experimental.pallas import tpu_sc as plsc

BATCH = 16384
VOCAB_N = 1000000
DIM = 64
LANES = 16
NUM_CORES = 2
NUM_SUBCORES = 16
NW = NUM_CORES * NUM_SUBCORES
SSHIFT = 15
STRIPE = 1 << SSHIFT
BLKW = 512
SWEEP_END = (VOCAB_N // BLKW) * BLKW
TAILW = 128
TAILB = VOCAB_N - TAILW
LASTW = TAILB >> SSHIFT
CAP = BATCH + 128

_mesh = plsc.VectorSubcoreMesh(core_axis_name="c", subcore_axis_name="s")


NSLOT = 6
PSHIFT = 14


def _sweep_window(buf, rb, width, mc, my_pk, pend_pk, colbuf, stag, sem_o,
                  st):
    lanes = lax.iota(jnp.int32, LANES)
    ones = jnp.full((LANES,), 1, jnp.int32)
    zeros = jnp.full((LANES,), 0, jnp.int32)
    nch4 = (mc + 4 * LANES - 1) // (4 * LANES)

    def scan4(k4, bc):
        for j in range(4):
            off = (k4 * 4 + j) * LANES
            pk = my_pk[pl.ds(off, LANES)]
            vr = pk >> PSHIFT
            valid = lanes < (mc - off)
            inb = valid & (vr >= rb) & (vr < rb + width)
            pref = plsc.cumsum(jnp.where(inb, ones, zeros))
            stgt = jnp.where(inb, bc + pref - 1, 0)
            plsc.store_scatter(pend_pk, [stgt], pk, mask=inb)
            bc = bc + plsc.all_reduce_population_count(inb)[0]
        return bc

    bc = lax.fori_loop(0, nch4, scan4, 0)
    ng = (bc + LANES - 1) // LANES

    def egroup(g, est):
        gctr, drained, issued = est
        ndrain = jnp.where((gctr > 0) & (lax.rem(gctr, NSLOT) == 0),
                           issued - drained, 0)

        def dwait(_l, c2):
            pltpu.make_async_copy(stag.at[pl.ds(0, 2 * DIM)],
                                  colbuf.at[pl.ds(0, 2 * DIM)], sem_o).wait()
            return c2

        lax.fori_loop(0, ndrain, dwait, 0)
        drained = drained + ndrain

        goff = g * LANES
        rem = bc - goff
        pk = pend_pk[pl.ds(goff, LANES)]
        cols = jnp.where(lanes < rem, (pk >> PSHIFT) - rb, 0)
        poss = pk & ((1 << PSHIFT) - 1)
        slot = lax.rem(gctr, NSLOT) * (LANES * 2 * DIM)
        for d in range(DIM):
            vals = plsc.load_gather(
                buf, [jnp.full((LANES,), d, jnp.int32), cols])
            plsc.store_scatter(colbuf, [slot + lanes * (2 * DIM) + d], vals)
        for l in range(LANES):
            @pl.when(l < rem)
            def _():
                pltpu.async_copy(
                    colbuf.at[pl.ds(slot + l * 2 * DIM, 2 * DIM)],
                    stag.at[pl.ds(poss[l] * (2 * DIM), 2 * DIM)], sem_o)
        issued = issued + jnp.minimum(rem, LANES)
        return gctr + 1, drained, issued

    return lax.fori_loop(0, ng, egroup, st)


def _serve(w, idx_hbm, tab_hbm, tail_hbm, stag, idxall, my_pk, pend_pk,
           blockbuf0, blockbuf1, colbuf, sem_o, sem_b):
    pltpu.sync_copy(idx_hbm, idxall)

    lanes = lax.iota(jnp.int32, LANES)
    ones = jnp.full((LANES,), 1, jnp.int32)
    zeros = jnp.full((LANES,), 0, jnp.int32)
    lo = w * STRIPE

    def fbody(k4, cnt):
        for j in range(4):
            k = k4 * 4 + j
            v = idxall[pl.ds(k * LANES, LANES)]
            m = (v >> SSHIFT) == w
            pk = ((v - lo) << PSHIFT) | (k * LANES + lanes)
            pref = plsc.cumsum(jnp.where(m, ones, zeros))
            stgt = jnp.where(m, cnt + pref - 1, 0)
            plsc.store_scatter(my_pk, [stgt], pk, mask=m)
            cnt = cnt + plsc.all_reduce_population_count(m)[0]
        return cnt

    mc = lax.fori_loop(0, BATCH // LANES // 4, fbody, 0)

    nb = jnp.where(w == LASTW, (SWEEP_END - LASTW * STRIPE) // BLKW,
                   jnp.where(w > LASTW, 0, STRIPE // BLKW))

    def _fetch(i, buf):
        tb = jnp.minimum(lo + i * BLKW, SWEEP_END - BLKW)
        pltpu.async_copy(tab_hbm.at[:, pl.ds(pl.multiple_of(tb, BLKW), BLKW)],
                         buf, sem_b)

    def _bwait(buf):
        pltpu.make_async_copy(tab_hbm.at[:, pl.ds(0, BLKW)], buf, sem_b).wait()

    _fetch(0, blockbuf0)

    def bbody(b2, st):
        i0 = 2 * b2
        i1 = i0 + 1
        _bwait(blockbuf0)
        _fetch(i1, blockbuf1)
        st = _sweep_window(blockbuf0, i0 * BLKW, BLKW, mc, my_pk, pend_pk,
                           colbuf, stag, sem_o, st)
        _bwait(blockbuf1)
        _fetch(i0 + 2, blockbuf0)
        w1 = jnp.where(i1 < nb, BLKW, 0)
        st = _sweep_window(blockbuf1, i1 * BLKW, w1, mc, my_pk, pend_pk,
                           colbuf, stag, sem_o, st)
        return st

    st = lax.fori_loop(0, (nb + 1) // 2, bbody, (0, 0, 0))
    _bwait(blockbuf0)

    @pl.when(w == LASTW)
    def _():
        pltpu.sync_copy(tail_hbm, blockbuf0.at[:, pl.ds(0, TAILW)])

    wt = jnp.where(w == LASTW, TAILW, 0)
    st = _sweep_window(blockbuf0, TAILB - LASTW * STRIPE, wt, mc, my_pk,
                       pend_pk, colbuf, stag, sem_o, st)

    _gctr, drained, issued = st

    def dwait2(_l, c2):
        pltpu.make_async_copy(stag.at[pl.ds(0, 2 * DIM)],
                              colbuf.at[pl.ds(0, 2 * DIM)], sem_o).wait()
        return c2

    lax.fori_loop(0, issued - drained, dwait2, 0)


@functools.partial(
    pl.kernel,
    mesh=_mesh,
    compiler_params=pltpu.CompilerParams(needs_layout_passes=False),
    out_type=(jax.ShapeDtypeStruct((BATCH * 2 * DIM,), jnp.float32),
              jax.ShapeDtypeStruct((BATCH * 2 * DIM,), jnp.float32)),
    scratch_types=[
        pltpu.VMEM((BATCH,), jnp.int32),
        pltpu.VMEM((CAP,), jnp.int32),
        pltpu.VMEM((BATCH,), jnp.int32),
        pltpu.VMEM((DIM, BLKW), jnp.float32),
        pltpu.VMEM((DIM, BLKW), jnp.float32),
        pltpu.VMEM((NSLOT * LANES * 2 * DIM,), jnp.float32),
        pltpu.SemaphoreType.DMA,
        pltpu.SemaphoreType.DMA,
    ],
)
def _gmf_gather(user_hbm, item_hbm, utab_hbm, itab_hbm, tail_u, tail_i,
                stag_u, stag_i,
                idxall, my_pk, pend_pk, blockbuf0,
                blockbuf1, colbuf, sem_o, sem_b):
    w = lax.axis_index("c") * NUM_SUBCORES + lax.axis_index("s")
    _serve(w, user_hbm, utab_hbm, tail_u, stag_u, idxall, my_pk, pend_pk,
           blockbuf0, blockbuf1, colbuf, sem_o, sem_b)
    _serve(w, item_hbm, itab_hbm, tail_i, stag_i, idxall, my_pk, pend_pk,
           blockbuf0, blockbuf1, colbuf, sem_o, sem_b)


_HB = 128
_RW = 2 * DIM


@functools.partial(
    pl.kernel,
    mesh=_mesh,
    compiler_params=pltpu.CompilerParams(needs_layout_passes=False),
    out_type=jax.ShapeDtypeStruct((BATCH * DIM,), jnp.float32),
    scratch_types=[
        pltpu.VMEM((_HB * _RW,), jnp.float32),
        pltpu.VMEM((_HB * _RW,), jnp.float32),
        pltpu.VMEM((_HB * DIM,), jnp.float32),
    ],
)
def _gmf_mul(su_hbm, si_hbm, out_hbm, ubuf, ibuf, obuf):
    wid = lax.axis_index("s") * NUM_CORES + lax.axis_index("c")
    base = wid * (BATCH // NW)

    for h in range(BATCH // NW // _HB):
        hb = base + h * _HB
        pltpu.sync_copy(su_hbm.at[pl.ds(hb * _RW, _HB * _RW)], ubuf)
        pltpu.sync_copy(si_hbm.at[pl.ds(hb * _RW, _HB * _RW)], ibuf)

        def rbody(r, carry):
            for g in range(DIM // LANES):
                so = pl.ds(r * _RW + g * LANES, LANES)
                do = pl.ds(r * DIM + g * LANES, LANES)
                obuf[do] = ubuf[so] * ibuf[so]
            return carry

        lax.fori_loop(0, _HB, rbody, 0)
        pltpu.sync_copy(obuf, out_hbm.at[pl.ds(hb * DIM, _HB * DIM)])


def kernel(user, item, user_table, item_table):
    tail_u = user_table[VOCAB_N - TAILW:].T
    tail_i = item_table[VOCAB_N - TAILW:].T
    su, si = _gmf_gather(user, item, user_table.T, item_table.T,
                         tail_u, tail_i)
    out1 = _gmf_mul(su, si)
    return out1.reshape(BATCH, DIM)


# speedup vs baseline: 2.9866x; 1.0085x over previous
import functools

import jax
import jax.numpy as jnp
from jax import lax
from jax.experimental import pallas as pl
from jax.experimental.pallas import tpu as pltpu
from jax.experimental.pallas import tpu_sc as plsc

BATCH = 16384
VOCAB_N = 1000000
DIM = 64
LANES = 16
NUM_CORES = 2
NUM_SUBCORES = 16
NW = NUM_CORES * NUM_SUBCORES
SSHIFT = 15
STRIPE = 1 << SSHIFT
BLKW = 512
SWEEP_END = (VOCAB_N // BLKW) * BLKW
TAILW = 128
TAILB = VOCAB_N - TAILW
LASTW = TAILB >> SSHIFT
CAP = BATCH + 128

_mesh = plsc.VectorSubcoreMesh(core_axis_name="c", subcore_axis_name="s")


NSLOT = 6
PSHIFT = 14


def _sweep_window(buf, rb, width, mc, my_pk, pend_pk, colbuf, stag, sem_o,
                  st):
    lanes = lax.iota(jnp.int32, LANES)
    ones = jnp.full((LANES,), 1, jnp.int32)
    zeros = jnp.full((LANES,), 0, jnp.int32)
    nch4 = (mc + 8 * LANES - 1) // (8 * LANES)

    def scan4(k4, bc):
        for j in range(8):
            off = (k4 * 8 + j) * LANES
            pk = my_pk[pl.ds(off, LANES)]
            vr = pk >> PSHIFT
            valid = lanes < (mc - off)
            inb = valid & (vr >= rb) & (vr < rb + width)
            pref = plsc.cumsum(jnp.where(inb, ones, zeros))
            stgt = jnp.where(inb, bc + pref - 1, 0)
            plsc.store_scatter(pend_pk, [stgt], pk, mask=inb)
            bc = bc + plsc.all_reduce_population_count(inb)[0]
        return bc

    bc = lax.fori_loop(0, nch4, scan4, 0)
    ng = (bc + LANES - 1) // LANES

    def egroup(g, est):
        gctr, drained, issued = est
        ndrain = jnp.where((gctr > 0) & (lax.rem(gctr, NSLOT) == 0),
                           issued - drained, 0)

        def dwait(_l, c2):
            pltpu.make_async_copy(stag.at[pl.ds(0, 2 * DIM)],
                                  colbuf.at[pl.ds(0, 2 * DIM)], sem_o).wait()
            return c2

        lax.fori_loop(0, ndrain, dwait, 0)
        drained = drained + ndrain

        goff = g * LANES
        rem = bc - goff
        pk = pend_pk[pl.ds(goff, LANES)]
        cols = jnp.where(lanes < rem, (pk >> PSHIFT) - rb, 0)
        poss = pk & ((1 << PSHIFT) - 1)
        slot = lax.rem(gctr, NSLOT) * (LANES * 2 * DIM)
        for d in range(DIM):
            vals = plsc.load_gather(
                buf, [jnp.full((LANES,), d, jnp.int32), cols])
            plsc.store_scatter(colbuf, [slot + lanes * (2 * DIM) + d], vals)
        for l in range(LANES):
            @pl.when(l < rem)
            def _():
                pltpu.async_copy(
                    colbuf.at[pl.ds(slot + l * 2 * DIM, 2 * DIM)],
                    stag.at[pl.ds(poss[l] * (2 * DIM), 2 * DIM)], sem_o)
        issued = issued + jnp.minimum(rem, LANES)
        return gctr + 1, drained, issued

    return lax.fori_loop(0, ng, egroup, st)


def _serve(w, idx_hbm, tab_hbm, tail_hbm, stag, idxall, my_pk, pend_pk,
           blockbuf0, blockbuf1, colbuf, sem_o, sem_b):
    pltpu.sync_copy(idx_hbm, idxall)

    lanes = lax.iota(jnp.int32, LANES)
    ones = jnp.full((LANES,), 1, jnp.int32)
    zeros = jnp.full((LANES,), 0, jnp.int32)
    lo = w * STRIPE

    def fbody(k4, cnt):
        for j in range(8):
            k = k4 * 8 + j
            v = idxall[pl.ds(k * LANES, LANES)]
            m = (v >> SSHIFT) == w
            pk = ((v - lo) << PSHIFT) | (k * LANES + lanes)
            pref = plsc.cumsum(jnp.where(m, ones, zeros))
            stgt = jnp.where(m, cnt + pref - 1, 0)
            plsc.store_scatter(my_pk, [stgt], pk, mask=m)
            cnt = cnt + plsc.all_reduce_population_count(m)[0]
        return cnt

    mc = lax.fori_loop(0, BATCH // LANES // 8, fbody, 0)

    nb = jnp.where(w == LASTW, (SWEEP_END - LASTW * STRIPE) // BLKW,
                   jnp.where(w > LASTW, 0, STRIPE // BLKW))

    def _fetch(i, buf):
        tb = jnp.minimum(lo + i * BLKW, SWEEP_END - BLKW)
        pltpu.async_copy(tab_hbm.at[:, pl.ds(pl.multiple_of(tb, BLKW), BLKW)],
                         buf, sem_b)

    def _bwait(buf):
        pltpu.make_async_copy(tab_hbm.at[:, pl.ds(0, BLKW)], buf, sem_b).wait()

    _fetch(0, blockbuf0)

    def bbody(b2, st):
        i0 = 2 * b2
        i1 = i0 + 1
        _bwait(blockbuf0)
        _fetch(i1, blockbuf1)
        st = _sweep_window(blockbuf0, i0 * BLKW, BLKW, mc, my_pk, pend_pk,
                           colbuf, stag, sem_o, st)
        _bwait(blockbuf1)
        _fetch(i0 + 2, blockbuf0)
        w1 = jnp.where(i1 < nb, BLKW, 0)
        st = _sweep_window(blockbuf1, i1 * BLKW, w1, mc, my_pk, pend_pk,
                           colbuf, stag, sem_o, st)
        return st

    st = lax.fori_loop(0, (nb + 1) // 2, bbody, (0, 0, 0))
    _bwait(blockbuf0)

    @pl.when(w == LASTW)
    def _():
        pltpu.sync_copy(tail_hbm, blockbuf0.at[:, pl.ds(0, TAILW)])

    wt = jnp.where(w == LASTW, TAILW, 0)
    st = _sweep_window(blockbuf0, TAILB - LASTW * STRIPE, wt, mc, my_pk,
                       pend_pk, colbuf, stag, sem_o, st)

    _gctr, drained, issued = st

    def dwait2(_l, c2):
        pltpu.make_async_copy(stag.at[pl.ds(0, 2 * DIM)],
                              colbuf.at[pl.ds(0, 2 * DIM)], sem_o).wait()
        return c2

    lax.fori_loop(0, issued - drained, dwait2, 0)


@functools.partial(
    pl.kernel,
    mesh=_mesh,
    compiler_params=pltpu.CompilerParams(needs_layout_passes=False),
    out_type=(jax.ShapeDtypeStruct((BATCH * 2 * DIM,), jnp.float32),
              jax.ShapeDtypeStruct((BATCH * 2 * DIM,), jnp.float32)),
    scratch_types=[
        pltpu.VMEM((BATCH,), jnp.int32),
        pltpu.VMEM((CAP,), jnp.int32),
        pltpu.VMEM((BATCH,), jnp.int32),
        pltpu.VMEM((DIM, BLKW), jnp.float32),
        pltpu.VMEM((DIM, BLKW), jnp.float32),
        pltpu.VMEM((NSLOT * LANES * 2 * DIM,), jnp.float32),
        pltpu.SemaphoreType.DMA,
        pltpu.SemaphoreType.DMA,
    ],
)
def _gmf_gather(user_hbm, item_hbm, utab_hbm, itab_hbm, tail_u, tail_i,
                stag_u, stag_i,
                idxall, my_pk, pend_pk, blockbuf0,
                blockbuf1, colbuf, sem_o, sem_b):
    w = lax.axis_index("c") * NUM_SUBCORES + lax.axis_index("s")
    _serve(w, user_hbm, utab_hbm, tail_u, stag_u, idxall, my_pk, pend_pk,
           blockbuf0, blockbuf1, colbuf, sem_o, sem_b)
    _serve(w, item_hbm, itab_hbm, tail_i, stag_i, idxall, my_pk, pend_pk,
           blockbuf0, blockbuf1, colbuf, sem_o, sem_b)


_HB = 128
_RW = 2 * DIM


@functools.partial(
    pl.kernel,
    mesh=_mesh,
    compiler_params=pltpu.CompilerParams(needs_layout_passes=False),
    out_type=jax.ShapeDtypeStruct((BATCH * DIM,), jnp.float32),
    scratch_types=[
        pltpu.VMEM((_HB * _RW,), jnp.float32),
        pltpu.VMEM((_HB * _RW,), jnp.float32),
        pltpu.VMEM((_HB * DIM,), jnp.float32),
    ],
)
def _gmf_mul(su_hbm, si_hbm, out_hbm, ubuf, ibuf, obuf):
    wid = lax.axis_index("s") * NUM_CORES + lax.axis_index("c")
    base = wid * (BATCH // NW)

    for h in range(BATCH // NW // _HB):
        hb = base + h * _HB
        pltpu.sync_copy(su_hbm.at[pl.ds(hb * _RW, _HB * _RW)], ubuf)
        pltpu.sync_copy(si_hbm.at[pl.ds(hb * _RW, _HB * _RW)], ibuf)

        def rbody(r, carry):
            for g in range(DIM // LANES):
                so = pl.ds(r * _RW + g * LANES, LANES)
                do = pl.ds(r * DIM + g * LANES, LANES)
                obuf[do] = ubuf[so] * ibuf[so]
            return carry

        lax.fori_loop(0, _HB, rbody, 0)
        pltpu.sync_copy(obuf, out_hbm.at[pl.ds(hb * DIM, _HB * DIM)])


def kernel(user, item, user_table, item_table):
    tail_u = user_table[VOCAB_N - TAILW:].T
    tail_i = item_table[VOCAB_N - TAILW:].T
    su, si = _gmf_gather(user, item, user_table.T, item_table.T,
                         tail_u, tail_i)
    out1 = _gmf_mul(su, si)
    return out1.reshape(BATCH, DIM)
